# baseline (speedup 1.0000x reference)
import jax
import jax.numpy as jnp
from jax import lax
from jax.experimental import pallas as pl
from jax.experimental.pallas import tpu as pltpu
from jax.experimental.pallas import tpu_sc as plsc

_B, _S, _D = 256, 4096, 64
_NC, _NS = 2, 16
_NW = _NC * _NS
_SG = 16
_RG = _NW // _SG
_SB = _S // _SG
_RB = _B // _RG
_L = 16


def _sc_body(x_hbm, embx_hbm, ids_hbm, copy_hbm, typed_hbm, feat_hbm,
             ids_v, embx_v, fbuf, xb0, xb1,
             si0, si1, sc0, sc1, st0, st1, sf0, sf1):
    wid = lax.axis_index("s") * _NC + lax.axis_index("c")
    s0 = (wid % _SG) * _SB
    r0 = (wid // _SG) * _RB

    pltpu.sync_copy(ids_hbm.at[pl.ds(s0, _SB)], ids_v)
    pltpu.sync_copy(embx_hbm, embx_v)

    def pat_body(d, _):
        for g in range(_SB // _L):
            sl = pl.ds(g * _L, _L)
            tv = ids_v[sl]
            fbuf[d, sl] = jnp.where(
                tv == 0, embx_v[d, pl.ds(0, _L)],
                jnp.where(tv == 1, embx_v[d, pl.ds(_L, _L)],
                          embx_v[d, pl.ds(2 * _L, _L)]))
        return 0

    lax.fori_loop(0, _D, pat_body, 0)

    mn = ids_v[pl.ds(0, _L)]
    mx = mn
    for g in range(1, _SB // _L):
        tv = ids_v[pl.ds(g * _L, _L)]
        mn = jnp.minimum(mn, tv)
        mx = jnp.maximum(mx, tv)
    t0 = mn[0]
    uniform = mx[0] == t0
    for k in range(1, _L):
        uniform = jnp.logical_and(
            uniform, jnp.logical_and(mn[k] == t0, mx[k] == t0))

    def xsl(r):
        return x_hbm.at[r, :, pl.ds(s0, _SB)]

    def add_chunk(xb):
        @pl.when(uniform)
        def _():
            def fast_body(d, _):
                pv = fbuf[d, pl.ds(0, _L)]
                for g in range(_SB // _L):
                    sl = pl.ds(g * _L, _L)
                    xb[d, sl] = xb[d, sl] + pv
                return 0
            lax.fori_loop(0, _D, fast_body, 0)

        @pl.when(jnp.logical_not(uniform))
        def _():
            def add_body(d, _):
                for g in range(_SB // _L):
                    sl = pl.ds(g * _L, _L)
                    xb[d, sl] = xb[d, sl] + fbuf[d, sl]
                return 0
            lax.fori_loop(0, _D, add_body, 0)

    pltpu.async_copy(xsl(r0), xb0, si0)

    def pair_body(i, _):
        a = r0 + 2 * i
        b = a + 1

        @pl.when(i > 0)
        def _():
            pltpu.make_async_copy(xsl(b), xb1, st1).wait()
        pltpu.async_copy(xsl(b), xb1, si1)

        pltpu.make_async_copy(xsl(a), xb0, si0).wait()
        d_co0 = pltpu.async_copy(xb0, copy_hbm.at[a, :, pl.ds(s0, _SB)], sc0)
        @pl.when(i > 0)
        def _():
            pltpu.make_async_copy(fbuf, feat_hbm.at[a, :, pl.ds(s0, _SB)], sf0).wait()
        pltpu.async_copy(fbuf, feat_hbm.at[a, :, pl.ds(s0, _SB)], sf0)
        d_co0.wait()
        add_chunk(xb0)
        pltpu.async_copy(xb0, typed_hbm.at[a, :, pl.ds(s0, _SB)], st0)

        pltpu.make_async_copy(xsl(b), xb1, si1).wait()
        d_co1 = pltpu.async_copy(xb1, copy_hbm.at[b, :, pl.ds(s0, _SB)], sc1)
        @pl.when(i > 0)
        def _():
            pltpu.make_async_copy(fbuf, feat_hbm.at[b, :, pl.ds(s0, _SB)], sf1).wait()
        pltpu.async_copy(fbuf, feat_hbm.at[b, :, pl.ds(s0, _SB)], sf1)

        pltpu.make_async_copy(xsl(a), xb0, st0).wait()
        @pl.when(i < _RB // 2 - 1)
        def _():
            pltpu.async_copy(xsl(a + 2), xb0, si0)

        d_co1.wait()
        add_chunk(xb1)
        pltpu.async_copy(xb1, typed_hbm.at[b, :, pl.ds(s0, _SB)], st1)
        return 0

    lax.fori_loop(0, _RB // 2, pair_body, 0)

    pltpu.make_async_copy(xsl(r0), xb1, st1).wait()
    pltpu.make_async_copy(fbuf, feat_hbm.at[r0, :, pl.ds(s0, _SB)], sf0).wait()
    pltpu.make_async_copy(fbuf, feat_hbm.at[r0, :, pl.ds(s0, _SB)], sf1).wait()


def kernel(slot_states, type_emb, slot_type_ids):
    B, S, D = slot_states.shape
    ids = slot_type_ids.astype(jnp.int32)
    xt = jnp.swapaxes(slot_states, 1, 2)
    embx = jnp.repeat(type_emb.T[:, :, None], _L, axis=2).reshape(D, 3 * _L)

    sc = pl.kernel(
        _sc_body,
        out_type=[jax.ShapeDtypeStruct((B, D, S), jnp.float32)] * 3,
        mesh=plsc.VectorSubcoreMesh(
            core_axis_name="c", subcore_axis_name="s",
            num_cores=_NC, num_subcores=_NS),
        scratch_types=[
            pltpu.VMEM((_SB,), jnp.int32),
            pltpu.VMEM((_D, 3 * _L), jnp.float32),
            pltpu.VMEM((_D, _SB), jnp.float32),
            pltpu.VMEM((_D, _SB), jnp.float32),
            pltpu.VMEM((_D, _SB), jnp.float32),
        ] + [pltpu.SemaphoreType.DMA] * 8,
    )
    copy_t, typed_t, feat_t = sc(xt, embx, ids)

    type_ids = jnp.broadcast_to(slot_type_ids[None, :], (B, S))
    slot_mask = jnp.ones((B, S), dtype=jnp.bool_)
    return (jnp.swapaxes(copy_t, 1, 2), jnp.swapaxes(typed_t, 1, 2),
            type_ids, jnp.swapaxes(feat_t, 1, 2), slot_mask)


# speedup vs baseline: 1.0297x; 1.0297x over previous
import jax
import jax.numpy as jnp
from jax import lax
from jax.experimental import pallas as pl
from jax.experimental.pallas import tpu as pltpu
from jax.experimental.pallas import tpu_sc as plsc

_B, _S, _D = 256, 4096, 64
_NC, _NS = 2, 16
_NW = _NC * _NS
_SG = 8
_RG = _NW // _SG
_SB = _S // _SG
_RB = _B // _RG
_L = 16


def _sc_body(x_hbm, embx_hbm, ids_hbm, copy_hbm, typed_hbm, feat_hbm,
             ids_v, embx_v, fbuf, xb0, xb1,
             si0, si1, sc0, sc1, st0, st1, sf0, sf1):
    wid = lax.axis_index("s") * _NC + lax.axis_index("c")
    s0 = (wid % _SG) * _SB
    r0 = (wid // _SG) * _RB

    pltpu.sync_copy(ids_hbm.at[pl.ds(s0, _SB)], ids_v)
    pltpu.sync_copy(embx_hbm, embx_v)

    def pat_body(d, _):
        for g in range(_SB // _L):
            sl = pl.ds(g * _L, _L)
            tv = ids_v[sl]
            fbuf[d, sl] = jnp.where(
                tv == 0, embx_v[d, pl.ds(0, _L)],
                jnp.where(tv == 1, embx_v[d, pl.ds(_L, _L)],
                          embx_v[d, pl.ds(2 * _L, _L)]))
        return 0

    lax.fori_loop(0, _D, pat_body, 0)

    mn = ids_v[pl.ds(0, _L)]
    mx = mn
    for g in range(1, _SB // _L):
        tv = ids_v[pl.ds(g * _L, _L)]
        mn = jnp.minimum(mn, tv)
        mx = jnp.maximum(mx, tv)
    t0 = mn[0]
    uniform = mx[0] == t0
    for k in range(1, _L):
        uniform = jnp.logical_and(
            uniform, jnp.logical_and(mn[k] == t0, mx[k] == t0))

    def xsl(r):
        return x_hbm.at[r, :, pl.ds(s0, _SB)]

    def add_chunk(xb):
        @pl.when(uniform)
        def _():
            def fast_body(d, _):
                pv = fbuf[d, pl.ds(0, _L)]
                for g in range(_SB // _L):
                    sl = pl.ds(g * _L, _L)
                    xb[d, sl] = xb[d, sl] + pv
                return 0
            lax.fori_loop(0, _D, fast_body, 0)

        @pl.when(jnp.logical_not(uniform))
        def _():
            def add_body(d, _):
                for g in range(_SB // _L):
                    sl = pl.ds(g * _L, _L)
                    xb[d, sl] = xb[d, sl] + fbuf[d, sl]
                return 0
            lax.fori_loop(0, _D, add_body, 0)

    pltpu.async_copy(xsl(r0), xb0, si0)

    def pair_body(i, _):
        a = r0 + 2 * i
        b = a + 1

        @pl.when(i > 0)
        def _():
            pltpu.make_async_copy(xsl(b), xb1, st1).wait()
        pltpu.async_copy(xsl(b), xb1, si1)

        pltpu.make_async_copy(xsl(a), xb0, si0).wait()
        d_co0 = pltpu.async_copy(xb0, copy_hbm.at[a, :, pl.ds(s0, _SB)], sc0)
        @pl.when(i > 0)
        def _():
            pltpu.make_async_copy(fbuf, feat_hbm.at[a, :, pl.ds(s0, _SB)], sf0).wait()
        pltpu.async_copy(fbuf, feat_hbm.at[a, :, pl.ds(s0, _SB)], sf0)
        d_co0.wait()
        add_chunk(xb0)
        pltpu.async_copy(xb0, typed_hbm.at[a, :, pl.ds(s0, _SB)], st0)

        pltpu.make_async_copy(xsl(b), xb1, si1).wait()
        d_co1 = pltpu.async_copy(xb1, copy_hbm.at[b, :, pl.ds(s0, _SB)], sc1)
        @pl.when(i > 0)
        def _():
            pltpu.make_async_copy(fbuf, feat_hbm.at[b, :, pl.ds(s0, _SB)], sf1).wait()
        pltpu.async_copy(fbuf, feat_hbm.at[b, :, pl.ds(s0, _SB)], sf1)

        pltpu.make_async_copy(xsl(a), xb0, st0).wait()
        @pl.when(i < _RB // 2 - 1)
        def _():
            pltpu.async_copy(xsl(a + 2), xb0, si0)

        d_co1.wait()
        add_chunk(xb1)
        pltpu.async_copy(xb1, typed_hbm.at[b, :, pl.ds(s0, _SB)], st1)
        return 0

    lax.fori_loop(0, _RB // 2, pair_body, 0)

    pltpu.make_async_copy(xsl(r0), xb1, st1).wait()
    pltpu.make_async_copy(fbuf, feat_hbm.at[r0, :, pl.ds(s0, _SB)], sf0).wait()
    pltpu.make_async_copy(fbuf, feat_hbm.at[r0, :, pl.ds(s0, _SB)], sf1).wait()


def kernel(slot_states, type_emb, slot_type_ids):
    B, S, D = slot_states.shape
    ids = slot_type_ids.astype(jnp.int32)
    xt = jnp.swapaxes(slot_states, 1, 2)
    embx = jnp.repeat(type_emb.T[:, :, None], _L, axis=2).reshape(D, 3 * _L)

    sc = pl.kernel(
        _sc_body,
        out_type=[jax.ShapeDtypeStruct((B, D, S), jnp.float32)] * 3,
        mesh=plsc.VectorSubcoreMesh(
            core_axis_name="c", subcore_axis_name="s",
            num_cores=_NC, num_subcores=_NS),
        scratch_types=[
            pltpu.VMEM((_SB,), jnp.int32),
            pltpu.VMEM((_D, 3 * _L), jnp.float32),
            pltpu.VMEM((_D, _SB), jnp.float32),
            pltpu.VMEM((_D, _SB), jnp.float32),
            pltpu.VMEM((_D, _SB), jnp.float32),
        ] + [pltpu.SemaphoreType.DMA] * 8,
    )
    copy_t, typed_t, feat_t = sc(xt, embx, ids)

    type_ids = jnp.broadcast_to(slot_type_ids[None, :], (B, S))
    slot_mask = jnp.ones((B, S), dtype=jnp.bool_)
    return (jnp.swapaxes(copy_t, 1, 2), jnp.swapaxes(typed_t, 1, 2),
            type_ids, jnp.swapaxes(feat_t, 1, 2), slot_mask)


# speedup vs baseline: 1.0520x; 1.0217x over previous
import jax
import jax.numpy as jnp
from jax import lax
from jax.experimental import pallas as pl
from jax.experimental.pallas import tpu as pltpu
from jax.experimental.pallas import tpu_sc as plsc

_B, _S, _D = 256, 4096, 64
_NC, _NS = 2, 16
_NW = _NC * _NS
_SG = 8
_RG = _NW // _SG
_SB = _S // _SG
_RB = _B // _RG
_L = 16


def _sc_body(x_hbm, embx_hbm, ids_hbm, copy_hbm, typed_hbm,
             ids_v, embx_v, fbuf, xb0, xb1,
             si0, si1, sc0, sc1, st0, st1):
    wid = lax.axis_index("s") * _NC + lax.axis_index("c")
    s0 = (wid % _SG) * _SB
    r0 = (wid // _SG) * _RB

    pltpu.sync_copy(ids_hbm.at[pl.ds(s0, _SB)], ids_v)
    pltpu.sync_copy(embx_hbm, embx_v)

    def pat_body(d, _):
        for g in range(_SB // _L):
            sl = pl.ds(g * _L, _L)
            tv = ids_v[sl]
            fbuf[d, sl] = jnp.where(
                tv == 0, embx_v[d, pl.ds(0, _L)],
                jnp.where(tv == 1, embx_v[d, pl.ds(_L, _L)],
                          embx_v[d, pl.ds(2 * _L, _L)]))
        return 0

    lax.fori_loop(0, _D, pat_body, 0)

    mn = ids_v[pl.ds(0, _L)]
    mx = mn
    for g in range(1, _SB // _L):
        tv = ids_v[pl.ds(g * _L, _L)]
        mn = jnp.minimum(mn, tv)
        mx = jnp.maximum(mx, tv)
    t0 = mn[0]
    uniform = mx[0] == t0
    for k in range(1, _L):
        uniform = jnp.logical_and(
            uniform, jnp.logical_and(mn[k] == t0, mx[k] == t0))

    def xsl(r):
        return x_hbm.at[r, :, pl.ds(s0, _SB)]

    def add_chunk(xb):
        @pl.when(uniform)
        def _():
            def fast_body(d, _):
                pv = fbuf[d, pl.ds(0, _L)]
                for g in range(_SB // _L):
                    sl = pl.ds(g * _L, _L)
                    xb[d, sl] = xb[d, sl] + pv
                return 0
            lax.fori_loop(0, _D, fast_body, 0)

        @pl.when(jnp.logical_not(uniform))
        def _():
            def add_body(d, _):
                for g in range(_SB // _L):
                    sl = pl.ds(g * _L, _L)
                    xb[d, sl] = xb[d, sl] + fbuf[d, sl]
                return 0
            lax.fori_loop(0, _D, add_body, 0)

    pltpu.async_copy(xsl(r0), xb0, si0)

    def pair_body(i, _):
        a = r0 + 2 * i
        b = a + 1

        @pl.when(i > 0)
        def _():
            pltpu.make_async_copy(xsl(b), xb1, st1).wait()
        pltpu.async_copy(xsl(b), xb1, si1)

        pltpu.make_async_copy(xsl(a), xb0, si0).wait()
        d_co0 = pltpu.async_copy(xb0, copy_hbm.at[a, :, pl.ds(s0, _SB)], sc0)
        d_co0.wait()
        add_chunk(xb0)
        pltpu.async_copy(xb0, typed_hbm.at[a, :, pl.ds(s0, _SB)], st0)

        pltpu.make_async_copy(xsl(b), xb1, si1).wait()
        d_co1 = pltpu.async_copy(xb1, copy_hbm.at[b, :, pl.ds(s0, _SB)], sc1)

        pltpu.make_async_copy(xsl(a), xb0, st0).wait()
        @pl.when(i < _RB // 2 - 1)
        def _():
            pltpu.async_copy(xsl(a + 2), xb0, si0)

        d_co1.wait()
        add_chunk(xb1)
        pltpu.async_copy(xb1, typed_hbm.at[b, :, pl.ds(s0, _SB)], st1)
        return 0

    lax.fori_loop(0, _RB // 2, pair_body, 0)

    pltpu.make_async_copy(xsl(r0), xb1, st1).wait()


def _tc_feat_body(ids_ref, embt_ref, feat_ref, pat_ref):
    @pl.when(pl.program_id(0) == 0)
    def _():
        tb = jnp.broadcast_to(ids_ref[...], (_D, _S))
        p = jnp.broadcast_to(embt_ref[:, 2:3], (_D, _S))
        for tt in (1, 0):
            p = jnp.where(tb == tt,
                          jnp.broadcast_to(embt_ref[:, tt:tt + 1], (_D, _S)),
                          p)
        pat_ref[...] = p
    feat_ref[0] = pat_ref[...]


def kernel(slot_states, type_emb, slot_type_ids):
    B, S, D = slot_states.shape
    ids = slot_type_ids.astype(jnp.int32)
    xt = jnp.swapaxes(slot_states, 1, 2)
    embx = jnp.repeat(type_emb.T[:, :, None], _L, axis=2).reshape(D, 3 * _L)

    sc = pl.kernel(
        _sc_body,
        out_type=[jax.ShapeDtypeStruct((B, D, S), jnp.float32)] * 2,
        mesh=plsc.VectorSubcoreMesh(
            core_axis_name="c", subcore_axis_name="s",
            num_cores=_NC, num_subcores=_NS),
        scratch_types=[
            pltpu.VMEM((_SB,), jnp.int32),
            pltpu.VMEM((_D, 3 * _L), jnp.float32),
            pltpu.VMEM((_D, _SB), jnp.float32),
            pltpu.VMEM((_D, _SB), jnp.float32),
            pltpu.VMEM((_D, _SB), jnp.float32),
        ] + [pltpu.SemaphoreType.DMA] * 6,
    )
    copy_t, typed_t = sc(xt, embx, ids)

    embt8 = jnp.concatenate(
        [type_emb.T, jnp.zeros((D, 8 - type_emb.shape[0]), type_emb.dtype)],
        axis=1)
    feat_t = pl.pallas_call(
        _tc_feat_body,
        grid=(B,),
        in_specs=[
            pl.BlockSpec((1, S), lambda b: (0, 0)),
            pl.BlockSpec((D, 8), lambda b: (0, 0)),
        ],
        out_specs=pl.BlockSpec((1, D, S), lambda b: (b, 0, 0)),
        out_shape=jax.ShapeDtypeStruct((B, D, S), jnp.float32),
        scratch_shapes=[pltpu.VMEM((_D, _S), jnp.float32)],
    )(ids.reshape(1, S), embt8)

    type_ids = jnp.broadcast_to(slot_type_ids[None, :], (B, S))
    slot_mask = jnp.ones((B, S), dtype=jnp.bool_)
    return (jnp.swapaxes(copy_t, 1, 2), jnp.swapaxes(typed_t, 1, 2),
            type_ids, jnp.swapaxes(feat_t, 1, 2), slot_mask)
